# baseline (speedup 1.0000x reference)
import jax
import jax.numpy as jnp
from jax import lax
from jax.experimental import pallas as pl
from jax.experimental.pallas import tpu as pltpu
from jax.experimental.pallas import tpu_sc as plsc

D_MODEL = 1024
TIME_STEPS = 16
MAX_LATENCY = 10

L = 16
NC = 2
NS = 16
NW = NC * NS
C = 2048
_RNE = 1.5 * 2.0**23


def _sc_body(x_hbm, out_hbm, xv, obuf, sem_in, sem_out):
    wid = lax.axis_index("s") * NC + lax.axis_index("c")

    zeros = jnp.zeros((L,), jnp.float32)
    ones = jnp.ones((L,), jnp.float32)

    def zinit(j, carry):
        for buf in range(2):
            for k in range(MAX_LATENCY + 1, TIME_STEPS):
                obuf[buf, pl.ds(k * C + j * L, L)] = zeros
        return carry

    lax.fori_loop(0, C // L, zinit, 0)

    N = x_hbm.shape[0]
    P = N // 2
    per_worker = N // NW
    n_chunks = per_worker // C
    base = wid * per_worker

    def in_copy(g, buf):
        q0 = pl.multiple_of(base + g * C, C)
        return pltpu.make_async_copy(
            x_hbm.at[pl.ds(q0, C)], xv.at[buf], sem_in.at[buf])

    def out_copy(g, buf):
        q0 = pl.multiple_of((base + g * C) * TIME_STEPS, C)
        return pltpu.make_async_copy(
            obuf.at[buf],
            out_hbm.at[pl.ds(q0, TIME_STEPS * C)],
            sem_out.at[buf])

    def compute(buf):
        def vec(j, carry):
            v = xv[buf, pl.ds(j * L, L)]
            s = 1.0 / (1.0 + jnp.exp(-v))
            y = MAX_LATENCY * (1.0 - s)
            t = (y + _RNE) - _RNE
            for k in range(MAX_LATENCY + 1):
                obuf[buf, k, pl.ds(j * L, L)] = jnp.where(
                    t == jnp.float32(k), ones, zeros)
            return carry

        lax.fori_loop(0, C // L, vec, 0)

    in_copy(0, 0).start()
    in_copy(1, 1).start()

    def pair(jj, carry):
        for buf in range(2):
            g = jj * 2 + buf
            in_copy(g, buf).wait()

            @pl.when(g >= 2)
            def _():
                out_copy(g - 2, buf).wait()

            out_copy(g, buf).start()

            @pl.when(g + 2 < n_chunks)
            def _():
                in_copy(g + 2, buf).start()

        return carry

    lax.fori_loop(0, n_chunks // 2, pair, 0)
    out_copy(n_chunks - 2, 0).wait()
    out_copy(n_chunks - 1, 1).wait()


def kernel(x):
    B, S, D = x.shape
    P = S * D
    xf = x.reshape(B * P)
    mesh = plsc.VectorSubcoreMesh(core_axis_name="c", subcore_axis_name="s")
    out = pl.kernel(
        _sc_body,
        mesh=mesh,
        out_type=jax.ShapeDtypeStruct((B * TIME_STEPS * P,), jnp.float32),
        scratch_types=[
            pltpu.VMEM((2, C), jnp.float32),
            pltpu.VMEM((2, TIME_STEPS * C), jnp.float32),
            pltpu.SemaphoreType.DMA((2,)),
            pltpu.SemaphoreType.DMA((2,)),
        ],
    )(xf)
    return out.reshape(B, TIME_STEPS, S, D)


# speedup vs baseline: 4.6061x; 4.6061x over previous
import jax
import jax.numpy as jnp
from jax.experimental import pallas as pl

D_MODEL = 1024
TIME_STEPS = 16
MAX_LATENCY = 10
BS = 128


def _body(x_ref, out_ref):
    xv = x_ref[0]
    t = jnp.round(MAX_LATENCY * (1.0 - jax.nn.sigmoid(xv)))
    for k in range(TIME_STEPS):
        out_ref[0, k] = jnp.where(t == jnp.float32(k), 1.0, 0.0).astype(jnp.float32)


def kernel(x):
    B, S, D = x.shape
    grid = (B, S // BS)
    return pl.pallas_call(
        _body,
        grid=grid,
        in_specs=[pl.BlockSpec((1, BS, D), lambda b, s: (b, s, 0))],
        out_specs=pl.BlockSpec((1, TIME_STEPS, BS, D), lambda b, s: (b, 0, s, 0)),
        out_shape=jax.ShapeDtypeStruct((B, TIME_STEPS, S, D), jnp.float32),
    )(x)
